# baseline (speedup 1.0000x reference)
import functools

import jax
import jax.numpy as jnp
from jax import lax
from jax.experimental import pallas as pl
from jax.experimental.pallas import tpu as pltpu
from jax.experimental.pallas import tpu_sc as plsc

PAD_ROW = 0
V = 1000
H = 64
B = 1024
L = 50
TOKENS = B * L

NC = 2
NS = 16
NW = NC * NS
PER_W = TOKENS // NW
CHUNK = 80
NCHUNK = PER_W // CHUNK


def _proj_body(emb_ref, w_ref, b_ref, out_ref):
    emb = emb_ref[:]
    rows = lax.broadcasted_iota(jnp.int32, emb.shape, 0)
    emb = jnp.where(rows == PAD_ROW, 0.0, emb)
    acc = lax.dot_general(
        emb, w_ref[:], (((1,), (1,)), ((), ())),
        preferred_element_type=jnp.float32,
    )
    out_ref[:] = acc + b_ref[:]


def _logit_table(emb, w, b):
    return pl.pallas_call(
        _proj_body,
        out_shape=jax.ShapeDtypeStruct((V, V), jnp.float32),
    )(emb, w, b.reshape(1, V))


def _gather_body(lt_hbm, ids_hbm, out_hbm, idx_v, rows_v, gsem):
    c = lax.axis_index("c")
    s = lax.axis_index("s")
    wid = s * NC + c
    base = wid * PER_W
    pltpu.sync_copy(ids_hbm.at[pl.ds(base, PER_W)], idx_v)

    def step(j, carry):
        cp = pltpu.async_copy(
            lt_hbm.at[idx_v.at[pl.ds(j * CHUNK, CHUNK)]], rows_v, gsem)
        cp.wait()
        pltpu.sync_copy(rows_v, out_hbm.at[pl.ds(base + j * CHUNK, CHUNK)])
        return carry

    lax.fori_loop(0, NCHUNK, step, 0)


_gather = functools.partial(
    pl.kernel,
    out_type=jax.ShapeDtypeStruct((TOKENS, V), jnp.float32),
    mesh=plsc.VectorSubcoreMesh(core_axis_name="c", subcore_axis_name="s"),
    compiler_params=pltpu.CompilerParams(use_tc_tiling_on_sc=False),
    scratch_types=[
        pltpu.VMEM((PER_W,), jnp.int32),
        pltpu.VMEM((CHUNK, V), jnp.float32),
        pltpu.SemaphoreType.DMA,
    ],
)(_gather_body)


def kernel(token_ids, emb_table, W, b):
    lt = _logit_table(emb_table, W, b)
    ids = token_ids.reshape(-1)
    out = _gather(lt, ids)
    return out.reshape(B, L, V)


# speedup vs baseline: 1.1214x; 1.1214x over previous
import functools

import jax
import jax.numpy as jnp
from jax import lax
from jax.experimental import pallas as pl
from jax.experimental.pallas import tpu as pltpu
from jax.experimental.pallas import tpu_sc as plsc

PAD_ROW = 0
V = 1000
H = 64
B = 1024
L = 50
TOKENS = B * L

NC = 2
NS = 16
NW = NC * NS
PER_W = TOKENS // NW
CHUNK = 32
NCHUNK = PER_W // CHUNK


def _proj_body(emb_ref, w_ref, b_ref, out_ref):
    emb = emb_ref[:]
    rows = lax.broadcasted_iota(jnp.int32, emb.shape, 0)
    emb = jnp.where(rows == PAD_ROW, 0.0, emb)
    acc = lax.dot_general(
        emb, w_ref[:], (((1,), (1,)), ((), ())),
        preferred_element_type=jnp.float32,
    )
    out_ref[:] = acc + b_ref[:]


def _logit_table(emb, w, b):
    return pl.pallas_call(
        _proj_body,
        out_shape=jax.ShapeDtypeStruct((V, V), jnp.float32),
    )(emb, w, b.reshape(1, V))


def _gather_body(lt_hbm, ids_hbm, out_hbm, idx_v, lt_sh, rows_v, gsem, osem):
    c = lax.axis_index("c")
    s = lax.axis_index("s")
    wid = s * NC + c
    base = wid * PER_W

    row0 = jnp.where(s == NS - 1, V - 64, s * 64)
    pltpu.sync_copy(lt_hbm.at[pl.ds(row0, 64)], lt_sh.at[pl.ds(row0, 64)])
    pltpu.sync_copy(ids_hbm.at[pl.ds(base, PER_W)], idx_v)
    plsc.subcore_barrier()

    def step(j, carry):
        p = lax.rem(j, 2)

        @pl.when(j >= 2)
        def _():
            pltpu.make_async_copy(
                rows_v.at[p], out_hbm.at[pl.ds(base, CHUNK)], osem).wait()

        pltpu.async_copy(
            lt_sh.at[idx_v.at[pl.ds(j * CHUNK, CHUNK)]],
            rows_v.at[p], gsem).wait()
        pltpu.async_copy(
            rows_v.at[p], out_hbm.at[pl.ds(base + j * CHUNK, CHUNK)], osem)
        return carry

    lax.fori_loop(0, NCHUNK, step, 0)
    for t in (NCHUNK - 2, NCHUNK - 1):
        pltpu.make_async_copy(
            rows_v.at[t % 2],
            out_hbm.at[pl.ds(base + t * CHUNK, CHUNK)], osem).wait()


_gather = functools.partial(
    pl.kernel,
    out_type=jax.ShapeDtypeStruct((TOKENS, V), jnp.float32),
    mesh=plsc.VectorSubcoreMesh(core_axis_name="c", subcore_axis_name="s"),
    compiler_params=pltpu.CompilerParams(use_tc_tiling_on_sc=False),
    scratch_types=[
        pltpu.VMEM((PER_W,), jnp.int32),
        pltpu.VMEM_SHARED((V, V), jnp.float32),
        pltpu.VMEM((2, CHUNK, V), jnp.float32),
        pltpu.SemaphoreType.DMA,
        pltpu.SemaphoreType.DMA,
    ],
)(_gather_body)


def kernel(token_ids, emb_table, W, b):
    lt = _logit_table(emb_table, W, b)
    ids = token_ids.reshape(-1)
    out = _gather(lt, ids)
    return out.reshape(B, L, V)
